# Initial kernel scaffold; baseline (speedup 1.0000x reference)
import jax
import jax.numpy as jnp
from jax.experimental import pallas as pl


def kernel(x, k_cache, v_cache, cache_pos):
    raise NotImplementedError("write your pallas kernel here")


# speedup vs baseline: 1.0164x; 1.0164x over previous
import jax
import jax.numpy as jnp
from jax.experimental import pallas as pl
from jax.experimental.pallas import tpu as pltpu

_ROWS = 512


def _body(pos_ref, x_ref, k_ref, v_ref, ox_ref, ok_ref, ov_ref):
    b = pl.program_id(0)
    r = pl.program_id(1)
    pos = pos_ref[0]
    row0 = r * _ROWS
    rows = row0 + jax.lax.broadcasted_iota(jnp.int32, (1, _ROWS, 1), 1)
    k_mask = jnp.logical_and(b == 1, rows == pos)
    ok_ref[...] = jnp.where(k_mask, 100.0, k_ref[...])
    v_mask = jnp.logical_and(b == 5, rows == pos + 5)
    ov_ref[...] = jnp.where(v_mask, 200.0, v_ref[...])

    @pl.when(r == 0)
    def _():
        ox_ref[...] = x_ref[...] * x_ref[...]


def kernel(x, k_cache, v_cache, cache_pos):
    B, S, D = k_cache.shape
    nb = S // _ROWS
    pos = jnp.asarray(cache_pos, jnp.int32).reshape(1)
    grid_spec = pltpu.PrefetchScalarGridSpec(
        num_scalar_prefetch=1,
        grid=(B, nb),
        in_specs=[
            pl.BlockSpec((1, 1, D), lambda b, r, pos: (b, 0, 0)),
            pl.BlockSpec((1, _ROWS, D), lambda b, r, pos: (b, r, 0)),
            pl.BlockSpec((1, _ROWS, D), lambda b, r, pos: (b, r, 0)),
        ],
        out_specs=[
            pl.BlockSpec((1, 1, D), lambda b, r, pos: (b, 0, 0)),
            pl.BlockSpec((1, _ROWS, D), lambda b, r, pos: (b, r, 0)),
            pl.BlockSpec((1, _ROWS, D), lambda b, r, pos: (b, r, 0)),
        ],
    )
    out_shape = [
        jax.ShapeDtypeStruct(x.shape, x.dtype),
        jax.ShapeDtypeStruct(k_cache.shape, k_cache.dtype),
        jax.ShapeDtypeStruct(v_cache.shape, v_cache.dtype),
    ]
    ox, ok, ov = pl.pallas_call(
        _body,
        grid_spec=grid_spec,
        out_shape=out_shape,
    )(pos, x, k_cache, v_cache)
    return (ox, ok, ov)
